# baseline (speedup 1.0000x reference)
import functools

import jax
import jax.numpy as jnp
from jax import lax
from jax.experimental import pallas as pl
from jax.experimental.pallas import tpu as pltpu
from jax.experimental.pallas import tpu_sc as plsc

EMB_DIM = 64
N_NEGS = 16384
GROUP = 8
GROUP_W = GROUP * EMB_DIM
NUM_CORES = 2
NUM_SUBCORES = 16
NUM_WORKERS = NUM_CORES * NUM_SUBCORES
B_PER_W = N_NEGS // NUM_WORKERS
BATCH = 64
N_BATCH = B_PER_W // BATCH


def _gather_body(emb8_hbm, negix_hbm, uvix_hbm, negs_out, uv_out,
                 idx_v, idxo_v, bufs, stage, uvidx_v, uvstage, sem):
    c = lax.axis_index("c")
    s = lax.axis_index("s")
    wid = s * NUM_CORES + c
    base = wid * B_PER_W
    lane = lax.iota(jnp.int32, 16)

    pltpu.sync_copy(negix_hbm.at[pl.ds(base, B_PER_W)], idx_v)

    def prep(k, _):
        x = idx_v[pl.ds(k * 16, 16)]
        idxo_v[pl.ds(k * 16, 16)] = x // GROUP
        return 0

    lax.fori_loop(0, B_PER_W // 16, prep, 0)

    def fire(b, slot):
        return pltpu.async_copy(
            emb8_hbm.at[idxo_v.at[pl.ds(b * BATCH, BATCH)]],
            bufs.at[slot], sem.at[slot])

    fire(0, 0)
    fire(1, 1)

    def extract_batch(b, slot):
        for hb in range(4):
            off = b * BATCH + hb * 16
            x = idx_v[pl.ds(off, 16)]
            sub = lax.rem(x, GROUP)
            rows = jnp.full((16,), hb * 16, jnp.int32) + lane

            def dstep(dq, _, sub=sub, rows=rows, slot=slot):
                for dd in range(8):
                    d = dq * 8 + dd
                    cols = sub * EMB_DIM + d
                    vals = plsc.load_gather(bufs.at[slot], [rows, cols])
                    plsc.store_scatter(
                        stage, [rows, jnp.full((16,), 1, jnp.int32) * d],
                        vals)
                return 0

            lax.fori_loop(0, 8, dstep, 0)
        pltpu.sync_copy(stage, negs_out.at[pl.ds(base + b * BATCH, BATCH)])

    def step(b, _):
        slot = lax.rem(b, 2)
        pltpu.make_async_copy(
            emb8_hbm.at[idxo_v.at[pl.ds(0, BATCH)]], bufs.at[slot],
            sem.at[slot]).wait()
        extract_batch(b, slot)

        @pl.when(b + 2 < N_BATCH)
        def _():
            fire(b + 2, slot)
        return 0

    lax.fori_loop(0, N_BATCH, step, 0)

    @pl.when(wid == 0)
    def _():
        pltpu.sync_copy(uvix_hbm, uvidx_v)
        x0 = uvidx_v[pl.ds(0, 16)]
        idxo_v[pl.ds(0, 16)] = x0 // GROUP
        pltpu.async_copy(emb8_hbm.at[idxo_v.at[pl.ds(0, 16)]],
                         bufs.at[0, pl.ds(0, 16)], sem.at[0]).wait()
        x = uvidx_v[pl.ds(0, 16)]
        sub = lax.rem(x, GROUP)

        def udstep(dq, _, sub=sub):
            for dd in range(8):
                d = dq * 8 + dd
                cols = sub * EMB_DIM + d
                vals = plsc.load_gather(bufs.at[0], [lane, cols])
                plsc.store_scatter(
                    uvstage, [lane, jnp.full((16,), 1, jnp.int32) * d], vals)
            return 0

        lax.fori_loop(0, 8, udstep, 0)
        pltpu.sync_copy(uvstage.at[pl.ds(0, 8)], uv_out)


_gather = functools.partial(
    pl.kernel,
    out_type=(
        jax.ShapeDtypeStruct((N_NEGS, EMB_DIM), jnp.float32),
        jax.ShapeDtypeStruct((8, EMB_DIM), jnp.float32),
    ),
    mesh=plsc.VectorSubcoreMesh(core_axis_name="c", subcore_axis_name="s"),
    compiler_params=pltpu.CompilerParams(
        use_tc_tiling_on_sc=False, needs_layout_passes=False),
    scratch_types=(
        pltpu.VMEM((B_PER_W,), jnp.int32),
        pltpu.VMEM((B_PER_W,), jnp.int32),
        pltpu.VMEM((2, BATCH, GROUP_W), jnp.float32),
        pltpu.VMEM((BATCH, EMB_DIM), jnp.float32),
        pltpu.VMEM((16,), jnp.int32),
        pltpu.VMEM((16, EMB_DIM), jnp.float32),
        pltpu.SemaphoreType.DMA((2,)),
    ),
)(_gather_body)


def _loss_body(negs_ref, uv_ref, out_ref):
    u = uv_ref[0:1, :]
    v = uv_ref[1:2, :]
    eps = 1e-5
    uu = jnp.sum(u * u)
    vv = jnp.sum(v * v)
    alpha = jnp.clip(1.0 - uu, eps, 1.0)
    beta_v = jnp.clip(1.0 - vv, eps, 1.0)
    sq_uv = jnp.sum((u - v) ** 2)
    gamma_uv = jnp.clip(1.0 + 2.0 * sq_uv / (alpha * beta_v), 1.0 + 1e-7, None)
    d_uv = jnp.log(gamma_uv + jnp.sqrt(gamma_uv * gamma_uv - 1.0))

    negs = negs_ref[...]
    nn = jnp.sum(negs * negs, axis=1, keepdims=True)
    beta_n = jnp.clip(1.0 - nn, eps, 1.0)
    sq_n = jnp.sum((negs - u) ** 2, axis=1, keepdims=True)
    gamma_n = jnp.clip(1.0 + 2.0 * sq_n / (alpha * beta_n), 1.0 + 1e-7, None)
    e_n = gamma_n - jnp.sqrt(gamma_n * gamma_n - 1.0)
    s_sum = jnp.sum(e_n)
    out_ref[...] = jnp.broadcast_to(d_uv + jnp.log(s_sum), (1, 1))


_loss = pl.pallas_call(
    _loss_body,
    out_shape=jax.ShapeDtypeStruct((1, 1), jnp.float32),
)


def kernel(embeddings, u_ix, v_ix, neg_ixs):
    neg_ixs = neg_ixs.astype(jnp.int32)
    u_ix = jnp.asarray(u_ix, jnp.int32)
    v_ix = jnp.asarray(v_ix, jnp.int32)
    uvix = jnp.stack([u_ix, v_ix] + [u_ix] * 14)
    emb8 = embeddings.reshape(1000000 // GROUP, GROUP_W)
    negs, uv = _gather(emb8, neg_ixs, uvix)
    loss = _loss(negs, uv)
    u = uv[0:1, :]
    v = uv[1:2, :]
    return (loss, u, v, negs)


# speedup vs baseline: 1.0664x; 1.0664x over previous
import functools

import jax
import jax.numpy as jnp
from jax import lax
from jax.experimental import pallas as pl
from jax.experimental.pallas import tpu as pltpu
from jax.experimental.pallas import tpu_sc as plsc

EMB_DIM = 64
N_NEGS = 16384
NUM_CORES = 2
NUM_SUBCORES = 16
NUM_WORKERS = NUM_CORES * NUM_SUBCORES
B_PER_W = N_NEGS // NUM_WORKERS
CHUNK = 128
N_CHUNKS = B_PER_W // CHUNK


def _gather_body(emb_hbm, negix_hbm, uvix_hbm, negs_out, uv_out,
                 idx_v, rows_v, uvidx_v, uvrows_v, sem):
    c = lax.axis_index("c")
    s = lax.axis_index("s")
    wid = s * NUM_CORES + c
    base = wid * B_PER_W
    pltpu.sync_copy(negix_hbm.at[pl.ds(base, B_PER_W)], idx_v)
    copies = [
        pltpu.async_copy(
            emb_hbm.at[idx_v.at[pl.ds(j * CHUNK, CHUNK)]],
            rows_v.at[pl.ds(j * CHUNK, CHUNK)],
            sem,
        )
        for j in range(N_CHUNKS)
    ]
    for cp in copies:
        cp.wait()
    pltpu.sync_copy(rows_v, negs_out.at[pl.ds(base, B_PER_W)])

    @pl.when(wid == 0)
    def _():
        pltpu.sync_copy(uvix_hbm, uvidx_v)
        pltpu.async_copy(emb_hbm.at[uvidx_v], uvrows_v, sem).wait()
        pltpu.sync_copy(uvrows_v, uv_out)


_gather = functools.partial(
    pl.kernel,
    out_type=(
        jax.ShapeDtypeStruct((N_NEGS, EMB_DIM), jnp.float32),
        jax.ShapeDtypeStruct((8, EMB_DIM), jnp.float32),
    ),
    mesh=plsc.VectorSubcoreMesh(core_axis_name="c", subcore_axis_name="s"),
    scratch_types=(
        pltpu.VMEM((B_PER_W,), jnp.int32),
        pltpu.VMEM((B_PER_W, EMB_DIM), jnp.float32),
        pltpu.VMEM((8,), jnp.int32),
        pltpu.VMEM((8, EMB_DIM), jnp.float32),
        pltpu.SemaphoreType.DMA,
    ),
    compiler_params=pltpu.CompilerParams(use_tc_tiling_on_sc=False),
)(_gather_body)


def _loss_body(negs_ref, uv_ref, out_ref):
    u = uv_ref[0:1, :]
    v = uv_ref[1:2, :]
    eps = 1e-5
    uu = jnp.sum(u * u)
    vv = jnp.sum(v * v)
    alpha = jnp.clip(1.0 - uu, eps, 1.0)
    beta_v = jnp.clip(1.0 - vv, eps, 1.0)
    sq_uv = jnp.sum((u - v) ** 2)
    gamma_uv = jnp.clip(1.0 + 2.0 * sq_uv / (alpha * beta_v), 1.0 + 1e-7, None)
    d_uv = jnp.log(gamma_uv + jnp.sqrt(gamma_uv * gamma_uv - 1.0))

    negs = negs_ref[...]
    nn = jnp.sum(negs * negs, axis=1, keepdims=True)
    beta_n = jnp.clip(1.0 - nn, eps, 1.0)
    sq_n = jnp.sum((negs - u) ** 2, axis=1, keepdims=True)
    gamma_n = jnp.clip(1.0 + 2.0 * sq_n / (alpha * beta_n), 1.0 + 1e-7, None)
    e_n = gamma_n - jnp.sqrt(gamma_n * gamma_n - 1.0)
    s_sum = jnp.sum(e_n)
    out_ref[...] = jnp.broadcast_to(d_uv + jnp.log(s_sum), (1, 1))


_loss = pl.pallas_call(
    _loss_body,
    out_shape=jax.ShapeDtypeStruct((1, 1), jnp.float32),
)


def kernel(embeddings, u_ix, v_ix, neg_ixs):
    neg_ixs = neg_ixs.astype(jnp.int32)
    u_ix = jnp.asarray(u_ix, jnp.int32)
    v_ix = jnp.asarray(v_ix, jnp.int32)
    uvix = jnp.stack([u_ix, v_ix, u_ix, u_ix, u_ix, u_ix, u_ix, u_ix])
    negs, uv = _gather(embeddings, neg_ixs, uvix)
    loss = _loss(negs, uv)
    u = uv[0:1, :]
    v = uv[1:2, :]
    return (loss, u, v, negs)
